# Initial kernel scaffold; baseline (speedup 1.0000x reference)
import jax
import jax.numpy as jnp
from jax.experimental import pallas as pl


def kernel(cond, emb):
    raise NotImplementedError("write your pallas kernel here")


# speedup vs baseline: 1.8099x; 1.8099x over previous
import jax
import jax.numpy as jnp
from jax.experimental import pallas as pl

ATTR_DIM = 26
N_OBJ = 20
EMBED = 64
BS = 1024
POS = ATTR_DIM * N_OBJ


def _body(cond_ref, emb_ref, out_ref):
    s = cond_ref[0] + cond_ref[1]
    bb = s.shape[0]
    obj = jnp.broadcast_to(emb_ref[...][None, :, :], (ATTR_DIM, N_OBJ, EMBED))
    obj = obj.reshape(POS, EMBED)
    obj = jnp.broadcast_to(obj[None, :, :], (bb, POS, EMBED))
    out_ref[...] = jnp.concatenate([s, obj], axis=-1)


def kernel(cond, emb):
    Bb = 16
    grid = (BS // Bb,)
    return pl.pallas_call(
        _body,
        grid=grid,
        in_specs=[
            pl.BlockSpec((2, Bb, POS, EMBED), lambda i: (0, i, 0, 0)),
            pl.BlockSpec((N_OBJ, EMBED), lambda i: (0, 0)),
        ],
        out_specs=pl.BlockSpec((Bb, POS, 2 * EMBED), lambda i: (i, 0, 0)),
        out_shape=jax.ShapeDtypeStruct((BS, POS, 2 * EMBED), jnp.float32),
    )(cond, emb)
